# baseline (speedup 1.0000x reference)
import functools

import jax
import jax.numpy as jnp
from jax import lax
from jax.experimental import pallas as pl
from jax.experimental.pallas import tpu as pltpu
from jax.experimental.pallas import tpu_sc as plsc

B = 16384
L = 200
D = 10
NC = 2
NS = 16
NW = NC * NS
BPW = B // NW
NGROUPS = BPW // 16

_mesh = plsc.VectorSubcoreMesh(
    core_axis_name="c", subcore_axis_name="s", num_cores=NC, num_subcores=NS
)


@functools.partial(
    pl.kernel,
    out_type=jax.ShapeDtypeStruct((D, B), jnp.float32),
    mesh=_mesh,
    scratch_types=[
        pltpu.VMEM((L, BPW), jnp.int32),
        pltpu.VMEM((D, BPW), jnp.float32),
        pltpu.VMEM((32,), jnp.float32),
        pltpu.SemaphoreType.DMA,
        pltpu.SemaphoreType.DMA,
    ],
    compiler_params=pltpu.CompilerParams(
        needs_layout_passes=False, use_tc_tiling_on_sc=True
    ),
)
def _bag_mean(xt_hbm, w_hbm, out_hbm, x_v, out_v, w_v, sem0, sem1):
    wid = lax.axis_index("s") * NC + lax.axis_index("c")
    b0 = wid * BPW
    half = BPW // 2
    cp0 = pltpu.async_copy(
        xt_hbm.at[:, pl.ds(b0, half)], x_v.at[:, pl.ds(0, half)], sem0
    )
    cp1 = pltpu.async_copy(
        xt_hbm.at[:, pl.ds(b0 + half, half)], x_v.at[:, pl.ds(half, half)], sem1
    )

    r0 = w_v[pl.ds(0, 16)]
    r1 = w_v[pl.ds(4, 16)]
    inv_l = jnp.float32(1.0 / L)
    w0b = []
    cfb = []
    for d in range(D):
        w0d = r0.at[jnp.full((16,), d, jnp.int32)].get(mode="promise_in_bounds")
        w1d = r1.at[jnp.full((16,), 6 + d, jnp.int32)].get(mode="promise_in_bounds")
        w0b.append(w0d)
        cfb.append((w1d - w0d) * inv_l)

    cp0.wait()

    def group_body(g, carry):
        pl.when(g == NGROUPS // 2)(cp1.wait)
        b = g * 16
        acc = jnp.zeros((16,), jnp.int32)
        for l in range(L):
            acc = acc + x_v[l, pl.ds(b, 16)]
        s = acc.astype(jnp.float32)
        for d in range(D):
            out_v[d, pl.ds(b, 16)] = w0b[d] + s * cfb[d]
        return carry

    lax.fori_loop(0, NGROUPS, group_body, 0)
    pltpu.sync_copy(out_v, out_hbm.at[:, pl.ds(b0, BPW)])


def kernel(x, use_quantized, W):
    del use_quantized
    xt = x.astype(jnp.int32).T
    wf = jnp.zeros((32,), jnp.float32).at[:20].set(W.astype(jnp.float32).reshape(20))
    out_t = _bag_mean(xt, wf)
    return out_t.T


# speedup vs baseline: 1.0135x; 1.0135x over previous
import functools

import jax
import jax.numpy as jnp
from jax import lax
from jax.experimental import pallas as pl
from jax.experimental.pallas import tpu as pltpu
from jax.experimental.pallas import tpu_sc as plsc

B = 16384
L = 200
D = 10
NC = 2
NS = 16
NW = NC * NS
BPW = B // NW
NGROUPS = BPW // 16

_mesh = plsc.VectorSubcoreMesh(
    core_axis_name="c", subcore_axis_name="s", num_cores=NC, num_subcores=NS
)


@functools.partial(
    pl.kernel,
    out_type=jax.ShapeDtypeStruct((D, B), jnp.float32),
    mesh=_mesh,
    scratch_types=[
        pltpu.VMEM((L, BPW), jnp.int32),
        pltpu.VMEM((D, BPW), jnp.float32),
        pltpu.VMEM((32,), jnp.float32),
        pltpu.SemaphoreType.DMA,
    ],
    compiler_params=pltpu.CompilerParams(
        needs_layout_passes=False, use_tc_tiling_on_sc=True
    ),
)
def _bag_mean(xt_hbm, w_hbm, out_hbm, x_v, out_v, w_v, sem):
    wid = lax.axis_index("s") * NC + lax.axis_index("c")
    b0 = wid * BPW
    cp = pltpu.async_copy(xt_hbm.at[:, pl.ds(b0, BPW)], x_v, sem)

    r0 = w_v[pl.ds(0, 16)]
    r1 = w_v[pl.ds(4, 16)]
    inv_l = jnp.float32(1.0 / L)
    w0b = []
    cfb = []
    for d in range(D):
        w0d = r0.at[jnp.full((16,), d, jnp.int32)].get(mode="promise_in_bounds")
        w1d = r1.at[jnp.full((16,), 6 + d, jnp.int32)].get(mode="promise_in_bounds")
        w0b.append(w0d)
        cfb.append((w1d - w0d) * inv_l)

    cp.wait()

    def group_body(g, carry):
        b = g * 16
        accs = [jnp.zeros((16,), jnp.int32) for _ in range(4)]
        for l in range(L):
            accs[l % 4] = accs[l % 4] + x_v[l, pl.ds(b, 16)]
        acc = (accs[0] + accs[1]) + (accs[2] + accs[3])
        s = acc.astype(jnp.float32)
        for d in range(D):
            out_v[d, pl.ds(b, 16)] = w0b[d] + s * cfb[d]
        return carry

    lax.fori_loop(0, NGROUPS, group_body, 0)
    pltpu.sync_copy(out_v, out_hbm.at[:, pl.ds(b0, BPW)])


def kernel(x, use_quantized, W):
    del use_quantized
    xt = x.astype(jnp.int32).T
    wf = jnp.zeros((32,), jnp.float32).at[:20].set(W.astype(jnp.float32).reshape(20))
    out_t = _bag_mean(xt, wf)
    return out_t.T
